# baseline (speedup 1.0000x reference)
import jax
import jax.numpy as jnp
from jax import lax
from jax.experimental import pallas as pl
from jax.experimental.pallas import tpu as pltpu
from jax.experimental.pallas import tpu_sc as plsc

_PC0, _PC1, _PC5 = -51.2, -51.2, 3.0
_GRID = 0.2
_POSW, _NEGW = 5.0, 0.1
_Y, _X = 512, 512
_B, _N = 4, 24
_RB = 64
_NR = _Y // _RB


def _raster_body(params_hbm, gt_hbm, params_v, gtbuf):
    c = lax.axis_index("c")
    s = lax.axis_index("s")
    wid = s * 2 + c
    b = wid // _NR
    slab = wid % _NR
    row0 = slab * _RB

    pltpu.sync_copy(params_hbm.at[b], params_v)

    zero16 = jnp.zeros((16,), jnp.float32)

    def _zero(t, _):
        gtbuf[t // 32, pl.ds(pl.multiple_of((t % 32) * 16, 16), 16)] = zero16
        return 0

    lax.fori_loop(0, _RB * 32, _zero, 0)

    lane = lax.iota(jnp.int32, 16).astype(jnp.float32)

    def _box(i, _):
        pv = params_v[i]
        cxg = pv[0]
        cyg = pv[1]
        cos_t = pv[2]
        sin_t = pv[3]
        hw = pv[4]
        hl = pv[5]
        hv = pv[6]
        ymin = pv[7]
        ymax = pv[8]
        xmin = pv[9]
        xmax = pv[10]

        y0 = jnp.maximum(ymin.astype(jnp.int32), row0)
        y1 = jnp.minimum(ymax.astype(jnp.int32), row0 + (_RB - 1))
        x0 = jnp.maximum(xmin.astype(jnp.int32), 0)
        x1 = jnp.minimum(xmax.astype(jnp.int32), _X - 1)
        cbase = jnp.minimum((x0 >> 4) << 4, _X - 48)
        hv_v = jnp.full((16,), hv, jnp.float32)

        def _row(y, _):
            dy = y.astype(jnp.float32) - cyg
            ys = dy * sin_t
            yc = dy * cos_t
            yl = y - row0
            for cc in range(3):
                cstart = pl.multiple_of(cbase + cc * 16, 16)
                dx = (lane + cstart.astype(jnp.float32)) - cxg
                l0 = dx * cos_t - ys
                l1 = dx * sin_t + yc
                inside = (jnp.abs(l0) <= hw) & (jnp.abs(l1) <= hl)
                old = gtbuf[yl, pl.ds(cstart, 16)]
                gtbuf[yl, pl.ds(cstart, 16)] = jnp.where(inside, hv_v, old)
            return 0

        lax.fori_loop(y0, y1 + 1, _row, 0)
        return 0

    lax.fori_loop(0, _N, _box, 0)

    pltpu.sync_copy(gtbuf, gt_hbm.at[b, pl.ds(row0, _RB)])


def _rasterize(params):
    mesh = plsc.VectorSubcoreMesh(
        core_axis_name="c", subcore_axis_name="s", num_cores=2, num_subcores=16
    )
    return pl.kernel(
        _raster_body,
        out_type=jax.ShapeDtypeStruct((_B, _Y, _X), jnp.float32),
        mesh=mesh,
        scratch_types=[
            pltpu.VMEM((_N, 16), jnp.float32),
            pltpu.VMEM((_RB, _X), jnp.float32),
        ],
    )(params)


def _loss_body(x_ref, gt_ref, hm_ref, out_ref, acc_ref):
    b = pl.program_id(0)
    r = pl.program_id(1)

    gt = gt_ref[0]
    x = x_ref[0, 0]
    hm = hm_ref[0, 0]

    pos = gt > 0.0
    weight = jnp.where(pos, _POSW, _NEGW)
    vf = (pos | (hm > 0.0)).astype(jnp.float32)

    bce = jnp.maximum(x, 0.0) - x * gt + jnp.log1p(jnp.exp(-jnp.abs(x)))
    p = jax.nn.sigmoid(x)
    p_t = p * gt + (1.0 - p) * (1.0 - gt)
    alpha_w = 0.25 * gt + 0.75 * (1.0 - gt)
    omp = 1.0 - p_t
    focal_w = omp * omp * alpha_w

    wb = weight * vf
    s_bce = jnp.sum(bce * wb)
    s_foc = jnp.sum(bce * focal_w * wb)
    s_cnt = jnp.sum(vf)

    @pl.when(r == 0)
    def _():
        acc_ref[b, 0] = 0.0
        acc_ref[b, 1] = 0.0
        acc_ref[b, 2] = 0.0

    acc_ref[b, 0] += s_bce
    acc_ref[b, 1] += s_foc
    acc_ref[b, 2] += s_cnt

    @pl.when(jnp.logical_and(b == _B - 1, r == _NR - 1))
    def _():
        total = jnp.float32(0.0)
        vs = jnp.float32(0.0)
        for bb in range(_B):
            cnt = acc_ref[bb, 2]
            denom = jnp.maximum(cnt, 1.0)
            comb = 0.5 * (acc_ref[bb, 0] + acc_ref[bb, 1]) / denom
            has_valid = (cnt > 0.0).astype(jnp.float32)
            total = total + comb * has_valid
            vs = vs + has_valid
        out_ref[0, 0] = jnp.where(vs > 0.0, total / jnp.maximum(vs, 1.0), total)


def _loss(attention_logits, gt, height_maps):
    return pl.pallas_call(
        _loss_body,
        grid=(_B, _NR),
        in_specs=[
            pl.BlockSpec((1, 1, _RB, _X), lambda b, r: (b, 0, r, 0)),
            pl.BlockSpec((1, _RB, _X), lambda b, r: (b, r, 0)),
            pl.BlockSpec((1, 1, _RB, _X), lambda b, r: (b, 0, r, 0)),
        ],
        out_specs=pl.BlockSpec(memory_space=pltpu.SMEM),
        out_shape=jax.ShapeDtypeStruct((1, 1), jnp.float32),
        scratch_shapes=[
            pltpu.SMEM((_B, 3), jnp.float32),
        ],
    )(attention_logits, gt, height_maps)


def _box_params(gt_bboxes_3d):
    cxg = (gt_bboxes_3d[..., 0] - _PC0) / _GRID
    cyg = (gt_bboxes_3d[..., 1] - _PC1) / _GRID
    wg2 = (gt_bboxes_3d[..., 3] / _GRID) / 2.0
    lg2 = (gt_bboxes_3d[..., 4] / _GRID) / 2.0
    theta = gt_bboxes_3d[..., 6]
    cos_t = jnp.cos(-theta)
    sin_t = jnp.sin(-theta)
    hv = gt_bboxes_3d[..., 5] / (_PC5 + 2.0)
    ey = jnp.abs(sin_t) * wg2 + jnp.abs(cos_t) * lg2
    ex = jnp.abs(cos_t) * wg2 + jnp.abs(sin_t) * lg2
    pad = jnp.zeros_like(cxg)
    return jnp.stack(
        [cxg, cyg, cos_t, sin_t, wg2, lg2, hv,
         cyg - ey, cyg + ey, cxg - ex, cxg + ex,
         pad, pad, pad, pad, pad], axis=-1
    )


def kernel(attention_logits, gt_bboxes_3d, height_maps):
    params = _box_params(gt_bboxes_3d)
    gt = _rasterize(params)
    return _loss(attention_logits, gt, height_maps)[0, 0]


# speedup vs baseline: 1.1964x; 1.1964x over previous
import jax
import jax.numpy as jnp
from jax import lax
from jax.experimental import pallas as pl
from jax.experimental.pallas import tpu as pltpu
from jax.experimental.pallas import tpu_sc as plsc

_PC0, _PC1, _PC5 = -51.2, -51.2, 3.0
_GRID = 0.2
_POSW, _NEGW = 5.0, 0.1
_Y, _X = 512, 512
_B, _N = 4, 24
_RB = 64
_NR = _Y // _RB


def _raster_body(params_hbm, gt_hbm, params_v, gtbuf):
    c = lax.axis_index("c")
    s = lax.axis_index("s")
    wid = s * 2 + c
    b = wid // _NR
    slab = wid % _NR
    row0 = slab * _RB

    pltpu.sync_copy(params_hbm.at[b], params_v)

    zero16 = jnp.zeros((16,), jnp.float32)

    def _zero(y, _):
        for j in range(_X // 16):
            gtbuf[y, pl.ds(j * 16, 16)] = zero16
        return 0

    lax.fori_loop(0, _RB, _zero, 0)

    lane = lax.iota(jnp.int32, 16).astype(jnp.float32)

    def _box(i, _):
        pv = params_v[i]
        cxg = pv[0]
        cyg = pv[1]
        cos_t = pv[2]
        sin_t = pv[3]
        hw = pv[4]
        hl = pv[5]
        hv = pv[6]
        ymin = pv[7]
        ymax = pv[8]
        xmin = pv[9]
        xmax = pv[10]

        y0 = jnp.maximum(ymin.astype(jnp.int32), row0)
        y1 = jnp.minimum(ymax.astype(jnp.int32), row0 + (_RB - 1))
        x0 = jnp.maximum(xmin.astype(jnp.int32), 0)
        x1 = jnp.minimum(xmax.astype(jnp.int32), _X - 1)
        cbase = jnp.minimum((x0 >> 4) << 4, _X - 48)
        hv_v = jnp.full((16,), hv, jnp.float32)

        def _row(y, _):
            dy = y.astype(jnp.float32) - cyg
            ys = dy * sin_t
            yc = dy * cos_t
            yl = y - row0
            for cc in range(3):
                cstart = pl.multiple_of(cbase + cc * 16, 16)
                dx = (lane + cstart.astype(jnp.float32)) - cxg
                l0 = dx * cos_t - ys
                l1 = dx * sin_t + yc
                inside = (jnp.abs(l0) <= hw) & (jnp.abs(l1) <= hl)
                old = gtbuf[yl, pl.ds(cstart, 16)]
                gtbuf[yl, pl.ds(cstart, 16)] = jnp.where(inside, hv_v, old)
            return 0

        lax.fori_loop(y0, y1 + 1, _row, 0)
        return 0

    lax.fori_loop(0, _N, _box, 0)

    pltpu.sync_copy(gtbuf, gt_hbm.at[b, pl.ds(row0, _RB)])


def _rasterize(params):
    mesh = plsc.VectorSubcoreMesh(
        core_axis_name="c", subcore_axis_name="s", num_cores=2, num_subcores=16
    )
    return pl.kernel(
        _raster_body,
        out_type=jax.ShapeDtypeStruct((_B, _Y, _X), jnp.float32),
        mesh=mesh,
        scratch_types=[
            pltpu.VMEM((_N, 16), jnp.float32),
            pltpu.VMEM((_RB, _X), jnp.float32),
        ],
    )(params)


_SL = 8


def _loss_body(x_ref, gt_ref, hm_ref, out_ref, acc_ref, vacc_ref):
    b = pl.program_id(0)
    r = pl.program_id(1)

    a_bce = jnp.zeros((_SL, _X), jnp.float32)
    a_foc = jnp.zeros((_SL, _X), jnp.float32)
    a_cnt = jnp.zeros((_SL, _X), jnp.float32)

    for k in range(_RB // _SL):
        sl = pl.ds(k * _SL, _SL)
        x = x_ref[sl, :]
        gt = gt_ref[sl, :]
        hm = hm_ref[sl, :]

        e = jnp.exp(-jnp.abs(x))
        bce0 = jnp.maximum(x, 0.0) + jnp.log1p(e)
        rp = 1.0 / (1.0 + e)
        p = jnp.where(x >= 0.0, rp, 1.0 - rp)

        pos = gt > 0.0
        point = hm > 0.0
        wb = jnp.where(pos, _POSW, jnp.where(point, _NEGW, 0.0))
        vf = jnp.where(pos | point, 1.0, 0.0)

        bce = bce0 - x * gt
        omp = p + gt * (1.0 - 2.0 * p)
        focal = omp * omp * (0.75 - 0.5 * gt)

        t1 = bce * wb
        a_bce = a_bce + t1
        a_foc = a_foc + t1 * focal
        a_cnt = a_cnt + vf

    @pl.when(r == 0)
    def _():
        vacc_ref[0:_SL] = a_bce
        vacc_ref[_SL:2 * _SL] = a_foc
        vacc_ref[2 * _SL:3 * _SL] = a_cnt

    @pl.when(r != 0)
    def _():
        vacc_ref[0:_SL] += a_bce
        vacc_ref[_SL:2 * _SL] += a_foc
        vacc_ref[2 * _SL:3 * _SL] += a_cnt

    @pl.when(r == _NR - 1)
    def _():
        acc_ref[b, 0] = jnp.sum(vacc_ref[0:_SL])
        acc_ref[b, 1] = jnp.sum(vacc_ref[_SL:2 * _SL])
        acc_ref[b, 2] = jnp.sum(vacc_ref[2 * _SL:3 * _SL])

    @pl.when(jnp.logical_and(b == _B - 1, r == _NR - 1))
    def _():
        total = jnp.float32(0.0)
        vs = jnp.float32(0.0)
        for bb in range(_B):
            cnt = acc_ref[bb, 2]
            denom = jnp.maximum(cnt, 1.0)
            comb = 0.5 * (acc_ref[bb, 0] + acc_ref[bb, 1]) / denom
            has_valid = (cnt > 0.0).astype(jnp.float32)
            total = total + comb * has_valid
            vs = vs + has_valid
        out_ref[0, 0] = jnp.where(vs > 0.0, total / jnp.maximum(vs, 1.0), total)


def _loss(attention_logits, gt, height_maps):
    x2 = attention_logits.reshape(_B * _Y, _X)
    gt2 = gt.reshape(_B * _Y, _X)
    hm2 = height_maps.reshape(_B * _Y, _X)
    return pl.pallas_call(
        _loss_body,
        grid=(_B, _NR),
        in_specs=[
            pl.BlockSpec((_RB, _X), lambda b, r: (b * _NR + r, 0)),
            pl.BlockSpec((_RB, _X), lambda b, r: (b * _NR + r, 0)),
            pl.BlockSpec((_RB, _X), lambda b, r: (b * _NR + r, 0)),
        ],
        out_specs=pl.BlockSpec(memory_space=pltpu.SMEM),
        out_shape=jax.ShapeDtypeStruct((1, 1), jnp.float32),
        scratch_shapes=[
            pltpu.SMEM((_B, 3), jnp.float32),
            pltpu.VMEM((3 * _SL, _X), jnp.float32),
        ],
    )(x2, gt2, hm2)


def _box_params(gt_bboxes_3d):
    cxg = (gt_bboxes_3d[..., 0] - _PC0) / _GRID
    cyg = (gt_bboxes_3d[..., 1] - _PC1) / _GRID
    wg2 = (gt_bboxes_3d[..., 3] / _GRID) / 2.0
    lg2 = (gt_bboxes_3d[..., 4] / _GRID) / 2.0
    theta = gt_bboxes_3d[..., 6]
    cos_t = jnp.cos(-theta)
    sin_t = jnp.sin(-theta)
    hv = gt_bboxes_3d[..., 5] / (_PC5 + 2.0)
    ey = jnp.abs(sin_t) * wg2 + jnp.abs(cos_t) * lg2
    ex = jnp.abs(cos_t) * wg2 + jnp.abs(sin_t) * lg2
    pad = jnp.zeros_like(cxg)
    return jnp.stack(
        [cxg, cyg, cos_t, sin_t, wg2, lg2, hv,
         cyg - ey, cyg + ey, cxg - ex, cxg + ex,
         pad, pad, pad, pad, pad], axis=-1
    )


def kernel(attention_logits, gt_bboxes_3d, height_maps):
    params = _box_params(gt_bboxes_3d)
    gt = _rasterize(params)
    return _loss(attention_logits, gt, height_maps)[0, 0]


# speedup vs baseline: 1.3094x; 1.0944x over previous
import jax
import jax.numpy as jnp
from jax import lax
from jax.experimental import pallas as pl
from jax.experimental.pallas import tpu as pltpu
from jax.experimental.pallas import tpu_sc as plsc

_PC0, _PC1, _PC5 = -51.2, -51.2, 3.0
_GRID = 0.2
_POSW, _NEGW = 5.0, 0.1
_Y, _X = 512, 512
_B, _N = 4, 24
_RB = 64
_NR = _Y // _RB
_PI = 3.14159265358979323846


def _sincos(th):
    n = th * (1.0 / _PI)
    k = (n + 0.5 * jnp.sign(n)).astype(jnp.int32)
    r = th - k.astype(jnp.float32) * _PI
    parity = (1 - 2 * (k & 1)).astype(jnp.float32)
    r2 = r * r
    sinp = r * (1.0 + r2 * (-1.0 / 6.0 + r2 * (1.0 / 120.0
                + r2 * (-1.0 / 5040.0 + r2 * (1.0 / 362880.0)))))
    cosp = 1.0 + r2 * (-1.0 / 2.0 + r2 * (1.0 / 24.0 + r2 * (-1.0 / 720.0
                + r2 * (1.0 / 40320.0 + r2 * (-1.0 / 3628800.0)))))
    return sinp * parity, cosp * parity


def _raster_body(boxes_hbm, zeros_hbm, gt_hbm, boxes_v, gtbuf, sem0, sem1):
    c = lax.axis_index("c")
    s = lax.axis_index("s")
    wid = s * 2 + c
    b = wid // _NR
    slab = wid % _NR
    row0 = slab * _RB

    zc = pltpu.async_copy(zeros_hbm, gtbuf, sem0)
    bc = pltpu.async_copy(boxes_hbm.at[b], boxes_v, sem1)
    bc.wait()
    zc.wait()

    lane = lax.iota(jnp.int32, 16).astype(jnp.float32)

    def _box(i, _):
        bv = boxes_v[i]
        sin_th, cos_th = _sincos(bv[6])
        cos_t = cos_th
        sin_t = -sin_th
        cxg = (bv[0] - _PC0) * (1.0 / _GRID)
        cyg = (bv[1] - _PC1) * (1.0 / _GRID)
        hw = bv[3] * (0.5 / _GRID)
        hl = bv[4] * (0.5 / _GRID)
        hv = bv[5] * (1.0 / (_PC5 + 2.0))
        asn = jnp.abs(sin_t)
        acs = jnp.abs(cos_t)
        ey = asn * hw + acs * hl
        ex = acs * hw + asn * hl

        y0 = jnp.maximum((cyg - ey).astype(jnp.int32), row0)
        y1 = jnp.minimum((cyg + ey).astype(jnp.int32), row0 + (_RB - 1))
        x0 = jnp.maximum((cxg - ex).astype(jnp.int32), 0)
        cbase = jnp.minimum((x0 >> 4) << 4, _X - 48)
        hv_v = jnp.full((16,), hv, jnp.float32)

        def _row(y, _):
            dy = y.astype(jnp.float32) - cyg
            ys = dy * sin_t
            yc = dy * cos_t
            yl = y - row0
            for cc in range(3):
                cstart = pl.multiple_of(cbase + cc * 16, 16)
                dx = (lane + cstart.astype(jnp.float32)) - cxg
                l0 = dx * cos_t - ys
                l1 = dx * sin_t + yc
                inside = (jnp.abs(l0) <= hw) & (jnp.abs(l1) <= hl)
                old = gtbuf[yl, pl.ds(cstart, 16)]
                gtbuf[yl, pl.ds(cstart, 16)] = jnp.where(inside, hv_v, old)
            return 0

        lax.fori_loop(y0, y1 + 1, _row, 0)
        return 0

    lax.fori_loop(0, _N, _box, 0)

    pltpu.sync_copy(gtbuf, gt_hbm.at[b, pl.ds(row0, _RB)])


def _rasterize(boxes_pad, zeros):
    mesh = plsc.VectorSubcoreMesh(
        core_axis_name="c", subcore_axis_name="s", num_cores=2, num_subcores=16
    )
    return pl.kernel(
        _raster_body,
        out_type=jax.ShapeDtypeStruct((_B, _Y, _X), jnp.float32),
        mesh=mesh,
        scratch_types=[
            pltpu.VMEM((_N, 16), jnp.float32),
            pltpu.VMEM((_RB, _X), jnp.float32),
            pltpu.SemaphoreType.DMA,
            pltpu.SemaphoreType.DMA,
        ],
    )(boxes_pad, zeros)


_RBT = 128
_NRT = _Y // _RBT
_SL = 8


def _loss_body(x_ref, gt_ref, hm_ref, out_ref, acc_ref, vacc_ref):
    b = pl.program_id(0)
    r = pl.program_id(1)

    a_bce = jnp.zeros((_SL, _X), jnp.float32)
    a_foc = jnp.zeros((_SL, _X), jnp.float32)
    a_cnt = jnp.zeros((_SL, _X), jnp.float32)

    for k in range(_RBT // _SL):
        sl = pl.ds(k * _SL, _SL)
        x = x_ref[sl, :]
        gt = gt_ref[sl, :]
        hm = hm_ref[sl, :]

        e = jnp.exp(-jnp.abs(x))
        bce0 = jnp.maximum(x, 0.0) + jnp.log1p(e)
        rp = 1.0 / (1.0 + e)
        p = jnp.where(x >= 0.0, rp, 1.0 - rp)

        pos = gt > 0.0
        point = hm > 0.0
        wb = jnp.where(pos, _POSW, jnp.where(point, _NEGW, 0.0))
        vf = jnp.where(pos | point, 1.0, 0.0)

        bce = bce0 - x * gt
        omp = p + gt * (1.0 - 2.0 * p)
        focal = omp * omp * (0.75 - 0.5 * gt)

        t1 = bce * wb
        a_bce = a_bce + t1
        a_foc = a_foc + t1 * focal
        a_cnt = a_cnt + vf

    @pl.when(r == 0)
    def _():
        vacc_ref[0:_SL] = a_bce
        vacc_ref[_SL:2 * _SL] = a_foc
        vacc_ref[2 * _SL:3 * _SL] = a_cnt

    @pl.when(r != 0)
    def _():
        vacc_ref[0:_SL] += a_bce
        vacc_ref[_SL:2 * _SL] += a_foc
        vacc_ref[2 * _SL:3 * _SL] += a_cnt

    @pl.when(r == _NRT - 1)
    def _():
        acc_ref[b, 0] = jnp.sum(vacc_ref[0:_SL])
        acc_ref[b, 1] = jnp.sum(vacc_ref[_SL:2 * _SL])
        acc_ref[b, 2] = jnp.sum(vacc_ref[2 * _SL:3 * _SL])

    @pl.when(jnp.logical_and(b == _B - 1, r == _NRT - 1))
    def _():
        total = jnp.float32(0.0)
        vs = jnp.float32(0.0)
        for bb in range(_B):
            cnt = acc_ref[bb, 2]
            denom = jnp.maximum(cnt, 1.0)
            comb = 0.5 * (acc_ref[bb, 0] + acc_ref[bb, 1]) / denom
            has_valid = (cnt > 0.0).astype(jnp.float32)
            total = total + comb * has_valid
            vs = vs + has_valid
        out_ref[0, 0] = jnp.where(vs > 0.0, total / jnp.maximum(vs, 1.0), total)


def _loss(attention_logits, gt, height_maps):
    x2 = attention_logits.reshape(_B * _Y, _X)
    gt2 = gt.reshape(_B * _Y, _X)
    hm2 = height_maps.reshape(_B * _Y, _X)
    return pl.pallas_call(
        _loss_body,
        grid=(_B, _NRT),
        in_specs=[
            pl.BlockSpec((_RBT, _X), lambda b, r: (b * _NRT + r, 0)),
            pl.BlockSpec((_RBT, _X), lambda b, r: (b * _NRT + r, 0)),
            pl.BlockSpec((_RBT, _X), lambda b, r: (b * _NRT + r, 0)),
        ],
        out_specs=pl.BlockSpec(memory_space=pltpu.SMEM),
        out_shape=jax.ShapeDtypeStruct((1, 1), jnp.float32),
        scratch_shapes=[
            pltpu.SMEM((_B, 3), jnp.float32),
            pltpu.VMEM((3 * _SL, _X), jnp.float32),
        ],
    )(x2, gt2, hm2)


def kernel(attention_logits, gt_bboxes_3d, height_maps):
    boxes_pad = jnp.pad(gt_bboxes_3d, ((0, 0), (0, 0), (0, 9)))
    zeros = jnp.zeros((_RB, _X), jnp.float32)
    gt = _rasterize(boxes_pad, zeros)
    return _loss(attention_logits, gt, height_maps)[0, 0]


# speedup vs baseline: 1.4213x; 1.0855x over previous
import jax
import jax.numpy as jnp
from jax import lax
from jax.experimental import pallas as pl
from jax.experimental.pallas import tpu as pltpu
from jax.experimental.pallas import tpu_sc as plsc

_PC0, _PC1, _PC5 = -51.2, -51.2, 3.0
_GRID = 0.2
_POSW, _NEGW = 5.0, 0.1
_Y, _X = 512, 512
_B, _N = 4, 24
_RB = 64
_NR = _Y // _RB
_PI = 3.14159265358979323846


def _sincos(th):
    n = th * (1.0 / _PI)
    k = (n + 0.5 * jnp.sign(n)).astype(jnp.int32)
    r = th - k.astype(jnp.float32) * _PI
    parity = (1 - 2 * (k & 1)).astype(jnp.float32)
    r2 = r * r
    sinp = r * (1.0 + r2 * (-1.0 / 6.0 + r2 * (1.0 / 120.0
                + r2 * (-1.0 / 5040.0 + r2 * (1.0 / 362880.0)))))
    cosp = 1.0 + r2 * (-1.0 / 2.0 + r2 * (1.0 / 24.0 + r2 * (-1.0 / 720.0
                + r2 * (1.0 / 40320.0 + r2 * (-1.0 / 3628800.0)))))
    return sinp * parity, cosp * parity


def _derive_params(boxes_v):
    chunks = []
    for c in range(2):
        sl = pl.ds(c * 16, 16)
        cx = boxes_v[0, sl]
        cy = boxes_v[1, sl]
        w = boxes_v[3, sl]
        l = boxes_v[4, sl]
        hh = boxes_v[5, sl]
        th = boxes_v[6, sl]
        sin_th, cos_th = _sincos(th)
        cos_t = cos_th
        sin_t = -sin_th
        cxg = (cx - _PC0) * (1.0 / _GRID)
        cyg = (cy - _PC1) * (1.0 / _GRID)
        hw = w * (0.5 / _GRID)
        hl = l * (0.5 / _GRID)
        hv = hh * (1.0 / (_PC5 + 2.0))
        asn = jnp.abs(sin_t)
        acs = jnp.abs(cos_t)
        ey = asn * hw + acs * hl
        ex = acs * hw + asn * hl
        chunks.append([cxg, cyg, cos_t, sin_t, hw, hl, hv,
                       cyg - ey, cyg + ey, cxg - ex, cxg + ex])
    return chunks


def _raster_body(boxes_hbm, gt_hbm, boxes_v, gtbuf):
    c = lax.axis_index("c")
    s = lax.axis_index("s")
    wid = s * 2 + c
    b = wid // _NR
    slab = wid % _NR
    row0 = slab * _RB

    pltpu.sync_copy(boxes_hbm.at[b], boxes_v)
    chunks = _derive_params(boxes_v)

    zero16 = jnp.zeros((16,), jnp.float32)

    def _zero(y, _):
        for j in range(_X // 16):
            gtbuf[y, pl.ds(j * 16, 16)] = zero16
        return 0

    lax.fori_loop(0, _RB, _zero, 0)

    lane = lax.iota(jnp.int32, 16).astype(jnp.float32)

    for i in range(_N):
        cv = chunks[i // 16]
        j = i % 16
        cxg = cv[0][j]
        cyg = cv[1][j]
        cos_t = cv[2][j]
        sin_t = cv[3][j]
        hw = cv[4][j]
        hl = cv[5][j]
        hv = cv[6][j]
        ymin = cv[7][j]
        ymax = cv[8][j]
        xmin = cv[9][j]
        xmax = cv[10][j]

        y0 = jnp.maximum(ymin.astype(jnp.int32), row0)
        y1 = jnp.minimum(ymax.astype(jnp.int32), row0 + (_RB - 1))
        x0 = jnp.maximum(xmin.astype(jnp.int32), 0)
        cbase = jnp.minimum((x0 >> 4) << 4, _X - 48)
        hv_v = jnp.full((16,), hv, jnp.float32)

        def _row(y, _, cxg=cxg, cyg=cyg, cos_t=cos_t, sin_t=sin_t,
                 hw=hw, hl=hl, hv_v=hv_v, cbase=cbase):
            dy = y.astype(jnp.float32) - cyg
            ys = dy * sin_t
            yc = dy * cos_t
            yl = y - row0
            for cc in range(3):
                cstart = pl.multiple_of(cbase + cc * 16, 16)
                dx = (lane + cstart.astype(jnp.float32)) - cxg
                l0 = dx * cos_t - ys
                l1 = dx * sin_t + yc
                inside = (jnp.abs(l0) <= hw) & (jnp.abs(l1) <= hl)
                old = gtbuf[yl, pl.ds(cstart, 16)]
                gtbuf[yl, pl.ds(cstart, 16)] = jnp.where(inside, hv_v, old)
            return 0

        lax.fori_loop(y0, y1 + 1, _row, 0)

    pltpu.sync_copy(gtbuf, gt_hbm.at[b, pl.ds(row0, _RB)])


def _rasterize(boxes_t):
    mesh = plsc.VectorSubcoreMesh(
        core_axis_name="c", subcore_axis_name="s", num_cores=2, num_subcores=16
    )
    return pl.kernel(
        _raster_body,
        out_type=jax.ShapeDtypeStruct((_B, _Y, _X), jnp.float32),
        mesh=mesh,
        scratch_types=[
            pltpu.VMEM((8, 32), jnp.float32),
            pltpu.VMEM((_RB, _X), jnp.float32),
        ],
    )(boxes_t)


_RBT = 256
_NRT = _Y // _RBT
_SL = 8


def _loss_body(x_ref, gt_ref, hm_ref, out_ref, acc_ref, vacc_ref):
    b = pl.program_id(0)
    r = pl.program_id(1)

    a_bce = jnp.zeros((_SL, _X), jnp.float32)
    a_foc = jnp.zeros((_SL, _X), jnp.float32)
    a_cnt = jnp.zeros((_SL, _X), jnp.float32)

    for k in range(_RBT // _SL):
        sl = pl.ds(k * _SL, _SL)
        x = x_ref[sl, :]
        gt = gt_ref[sl, :]
        hm = hm_ref[sl, :]

        e = jnp.exp(-jnp.abs(x))
        bce0 = jnp.maximum(x, 0.0) + jnp.log1p(e)
        rp = 1.0 / (1.0 + e)
        p = jnp.where(x >= 0.0, rp, 1.0 - rp)

        pos = gt > 0.0
        point = hm > 0.0
        wb = jnp.where(pos, _POSW, jnp.where(point, _NEGW, 0.0))
        vf = jnp.where(pos | point, 1.0, 0.0)

        bce = bce0 - x * gt
        omp = p + gt * (1.0 - 2.0 * p)
        focal = omp * omp * (0.75 - 0.5 * gt)

        t1 = bce * wb
        a_bce = a_bce + t1
        a_foc = a_foc + t1 * focal
        a_cnt = a_cnt + vf

    @pl.when(r == 0)
    def _():
        vacc_ref[0:_SL] = a_bce
        vacc_ref[_SL:2 * _SL] = a_foc
        vacc_ref[2 * _SL:3 * _SL] = a_cnt

    @pl.when(r != 0)
    def _():
        vacc_ref[0:_SL] += a_bce
        vacc_ref[_SL:2 * _SL] += a_foc
        vacc_ref[2 * _SL:3 * _SL] += a_cnt

    @pl.when(r == _NRT - 1)
    def _():
        acc_ref[b, 0] = jnp.sum(vacc_ref[0:_SL])
        acc_ref[b, 1] = jnp.sum(vacc_ref[_SL:2 * _SL])
        acc_ref[b, 2] = jnp.sum(vacc_ref[2 * _SL:3 * _SL])

    @pl.when(jnp.logical_and(b == _B - 1, r == _NRT - 1))
    def _():
        total = jnp.float32(0.0)
        vs = jnp.float32(0.0)
        for bb in range(_B):
            cnt = acc_ref[bb, 2]
            denom = jnp.maximum(cnt, 1.0)
            comb = 0.5 * (acc_ref[bb, 0] + acc_ref[bb, 1]) / denom
            has_valid = (cnt > 0.0).astype(jnp.float32)
            total = total + comb * has_valid
            vs = vs + has_valid
        out_ref[0, 0] = jnp.where(vs > 0.0, total / jnp.maximum(vs, 1.0), total)


def _loss(attention_logits, gt, height_maps):
    x2 = attention_logits.reshape(_B * _Y, _X)
    gt2 = gt.reshape(_B * _Y, _X)
    hm2 = height_maps.reshape(_B * _Y, _X)
    return pl.pallas_call(
        _loss_body,
        grid=(_B, _NRT),
        in_specs=[
            pl.BlockSpec((_RBT, _X), lambda b, r: (b * _NRT + r, 0)),
            pl.BlockSpec((_RBT, _X), lambda b, r: (b * _NRT + r, 0)),
            pl.BlockSpec((_RBT, _X), lambda b, r: (b * _NRT + r, 0)),
        ],
        out_specs=pl.BlockSpec(memory_space=pltpu.SMEM),
        out_shape=jax.ShapeDtypeStruct((1, 1), jnp.float32),
        scratch_shapes=[
            pltpu.SMEM((_B, 3), jnp.float32),
            pltpu.VMEM((3 * _SL, _X), jnp.float32),
        ],
    )(x2, gt2, hm2)


def kernel(attention_logits, gt_bboxes_3d, height_maps):
    boxes_t = jnp.pad(jnp.transpose(gt_bboxes_3d, (0, 2, 1)),
                      ((0, 0), (0, 1), (0, 8)))
    gt = _rasterize(boxes_t)
    return _loss(attention_logits, gt, height_maps)[0, 0]


# speedup vs baseline: 1.4577x; 1.0256x over previous
import jax
import jax.numpy as jnp
from jax import lax
from jax.experimental import pallas as pl
from jax.experimental.pallas import tpu as pltpu
from jax.experimental.pallas import tpu_sc as plsc

_PC0, _PC1, _PC5 = -51.2, -51.2, 3.0
_GRID = 0.2
_POSW, _NEGW = 5.0, 0.1
_Y, _X = 512, 512
_B, _N = 4, 24
_RB = 64
_NR = _Y // _RB
_PI = 3.14159265358979323846


def _sincos(th):
    n = th * (1.0 / _PI)
    k = (n + 0.5 * jnp.sign(n)).astype(jnp.int32)
    r = th - k.astype(jnp.float32) * _PI
    parity = (1 - 2 * (k & 1)).astype(jnp.float32)
    r2 = r * r
    sinp = r * (1.0 + r2 * (-1.0 / 6.0 + r2 * (1.0 / 120.0
                + r2 * (-1.0 / 5040.0 + r2 * (1.0 / 362880.0
                + r2 * (-1.0 / 39916800.0))))))
    cosp = 1.0 + r2 * (-1.0 / 2.0 + r2 * (1.0 / 24.0 + r2 * (-1.0 / 720.0
                + r2 * (1.0 / 40320.0 + r2 * (-1.0 / 3628800.0
                + r2 * (1.0 / 479001600.0))))))
    return sinp * parity, cosp * parity


def _derive_params(boxes_v):
    chunks = []
    for c in range(2):
        sl = pl.ds(c * 16, 16)
        cx = boxes_v[0, sl]
        cy = boxes_v[1, sl]
        w = boxes_v[3, sl]
        l = boxes_v[4, sl]
        hh = boxes_v[5, sl]
        th = boxes_v[6, sl]
        sin_th, cos_th = _sincos(th)
        cos_t = cos_th
        sin_t = -sin_th
        cxg = (cx - _PC0) * (1.0 / _GRID)
        cyg = (cy - _PC1) * (1.0 / _GRID)
        hw = w * (0.5 / _GRID)
        hl = l * (0.5 / _GRID)
        hv = hh * (1.0 / (_PC5 + 2.0))
        asn = jnp.abs(sin_t)
        acs = jnp.abs(cos_t)
        ey = asn * hw + acs * hl
        ex = acs * hw + asn * hl
        chunks.append([cxg, cyg, cos_t, sin_t, hw, hl, hv,
                       cyg - ey, cyg + ey, cxg - ex, cxg + ex])
    return chunks


def _raster_body(boxes_hbm, gt_hbm, boxes_v, gtbuf):
    c = lax.axis_index("c")
    s = lax.axis_index("s")
    wid = s * 2 + c
    b = wid // _NR
    slab = wid % _NR
    row0 = slab * _RB

    pltpu.sync_copy(boxes_hbm.at[b], boxes_v)
    chunks = _derive_params(boxes_v)

    zero16 = jnp.zeros((16,), jnp.float32)

    def _zero(y, _):
        for j in range(_X // 16):
            gtbuf[y, pl.ds(j * 16, 16)] = zero16
        return 0

    lax.fori_loop(0, _RB, _zero, 0)

    lane = lax.iota(jnp.int32, 16).astype(jnp.float32)

    for i in range(_N):
        cv = chunks[i // 16]
        j = i % 16
        cxg = cv[0][j]
        cyg = cv[1][j]
        cos_t = cv[2][j]
        sin_t = cv[3][j]
        hw = cv[4][j]
        hl = cv[5][j]
        hv = cv[6][j]
        ymin = cv[7][j]
        ymax = cv[8][j]
        xmin = cv[9][j]
        xmax = cv[10][j]

        y0 = jnp.maximum(ymin.astype(jnp.int32), row0)
        y1 = jnp.minimum(ymax.astype(jnp.int32), row0 + (_RB - 1))
        x0 = jnp.maximum(xmin.astype(jnp.int32), 0)
        cbase = jnp.minimum((x0 >> 4) << 4, _X - 48)
        hv_v = jnp.full((16,), hv, jnp.float32)

        def _row(y, _, cxg=cxg, cyg=cyg, cos_t=cos_t, sin_t=sin_t,
                 hw=hw, hl=hl, hv_v=hv_v, cbase=cbase):
            dy = y.astype(jnp.float32) - cyg
            ys = dy * sin_t
            yc = dy * cos_t
            yl = y - row0
            for cc in range(3):
                cstart = pl.multiple_of(cbase + cc * 16, 16)
                dx = (lane + cstart.astype(jnp.float32)) - cxg
                l0 = dx * cos_t - ys
                l1 = dx * sin_t + yc
                inside = (jnp.abs(l0) <= hw) & (jnp.abs(l1) <= hl)
                old = gtbuf[yl, pl.ds(cstart, 16)]
                gtbuf[yl, pl.ds(cstart, 16)] = jnp.where(inside, hv_v, old)
            return 0

        lax.fori_loop(y0, y1 + 1, _row, 0)

    pltpu.sync_copy(gtbuf, gt_hbm.at[b, pl.ds(row0, _RB)])


def _rasterize(boxes_t):
    mesh = plsc.VectorSubcoreMesh(
        core_axis_name="c", subcore_axis_name="s", num_cores=2, num_subcores=16
    )
    return pl.kernel(
        _raster_body,
        out_type=jax.ShapeDtypeStruct((_B, _Y, _X), jnp.float32),
        mesh=mesh,
        scratch_types=[
            pltpu.VMEM((8, 32), jnp.float32),
            pltpu.VMEM((_RB, _X), jnp.float32),
        ],
    )(boxes_t)


_RBT = 512
_NRT = _Y // _RBT
_SL = 8


def _loss_body(x_ref, gt_ref, hm_ref, out_ref, acc_ref, vacc_ref):
    b = pl.program_id(0)
    r = pl.program_id(1)

    a_bce = jnp.zeros((_SL, _X), jnp.float32)
    a_foc = jnp.zeros((_SL, _X), jnp.float32)
    a_cnt = jnp.zeros((_SL, _X), jnp.float32)

    for k in range(_RBT // _SL):
        sl = pl.ds(k * _SL, _SL)
        x = x_ref[sl, :]
        gt = gt_ref[sl, :]
        hm = hm_ref[sl, :]

        e = jnp.exp(-jnp.abs(x))
        bce0 = jnp.maximum(x, 0.0) + jnp.log1p(e)
        rp = 1.0 / (1.0 + e)
        p = jnp.where(x >= 0.0, rp, 1.0 - rp)

        pos = gt > 0.0
        point = hm > 0.0
        wb = jnp.where(pos, _POSW, jnp.where(point, _NEGW, 0.0))
        vf = jnp.where(pos | point, 1.0, 0.0)

        bce = bce0 - x * gt
        omp = p + gt * (1.0 - 2.0 * p)
        focal = omp * omp * (0.75 - 0.5 * gt)

        t1 = bce * wb
        a_bce = a_bce + t1
        a_foc = a_foc + t1 * focal
        a_cnt = a_cnt + vf

    @pl.when(r == 0)
    def _():
        vacc_ref[0:_SL] = a_bce
        vacc_ref[_SL:2 * _SL] = a_foc
        vacc_ref[2 * _SL:3 * _SL] = a_cnt

    @pl.when(r != 0)
    def _():
        vacc_ref[0:_SL] += a_bce
        vacc_ref[_SL:2 * _SL] += a_foc
        vacc_ref[2 * _SL:3 * _SL] += a_cnt

    @pl.when(r == _NRT - 1)
    def _():
        acc_ref[b, 0] = jnp.sum(vacc_ref[0:_SL])
        acc_ref[b, 1] = jnp.sum(vacc_ref[_SL:2 * _SL])
        acc_ref[b, 2] = jnp.sum(vacc_ref[2 * _SL:3 * _SL])

    @pl.when(jnp.logical_and(b == _B - 1, r == _NRT - 1))
    def _():
        total = jnp.float32(0.0)
        vs = jnp.float32(0.0)
        for bb in range(_B):
            cnt = acc_ref[bb, 2]
            denom = jnp.maximum(cnt, 1.0)
            comb = 0.5 * (acc_ref[bb, 0] + acc_ref[bb, 1]) / denom
            has_valid = (cnt > 0.0).astype(jnp.float32)
            total = total + comb * has_valid
            vs = vs + has_valid
        out_ref[0, 0] = jnp.where(vs > 0.0, total / jnp.maximum(vs, 1.0), total)


def _loss(attention_logits, gt, height_maps):
    x2 = attention_logits.reshape(_B * _Y, _X)
    gt2 = gt.reshape(_B * _Y, _X)
    hm2 = height_maps.reshape(_B * _Y, _X)
    return pl.pallas_call(
        _loss_body,
        grid=(_B, _NRT),
        in_specs=[
            pl.BlockSpec((_RBT, _X), lambda b, r: (b * _NRT + r, 0)),
            pl.BlockSpec((_RBT, _X), lambda b, r: (b * _NRT + r, 0)),
            pl.BlockSpec((_RBT, _X), lambda b, r: (b * _NRT + r, 0)),
        ],
        out_specs=pl.BlockSpec(memory_space=pltpu.SMEM),
        out_shape=jax.ShapeDtypeStruct((1, 1), jnp.float32),
        scratch_shapes=[
            pltpu.SMEM((_B, 3), jnp.float32),
            pltpu.VMEM((3 * _SL, _X), jnp.float32),
        ],
    )(x2, gt2, hm2)


def kernel(attention_logits, gt_bboxes_3d, height_maps):
    boxes_t = jnp.pad(jnp.transpose(gt_bboxes_3d, (0, 2, 1)),
                      ((0, 0), (0, 1), (0, 8)))
    gt = _rasterize(boxes_t)
    return _loss(attention_logits, gt, height_maps)[0, 0]
